# baseline (speedup 1.0000x reference)
import functools

import jax
import jax.numpy as jnp
from jax import lax
from jax.experimental import pallas as pl
from jax.experimental.pallas import tpu as pltpu
from jax.experimental.pallas import tpu_sc as plsc

N = 10000
E = 320000
D = 128
NP = 10240
NC = 2
NS = 16
NW = NC * NS
EW = E // NW
K = 40
NB = EW // K
SBB = 50
NSB = NB // SBB
KE = 80
NT = NP // NS
RW = NP // NW
LANES = 16

_MESH = plsc.VectorSubcoreMesh(core_axis_name="c", subcore_axis_name="s")


def _fill_rows(ref, nrows, ncols, val):
    def row(i, carry):
        def col(k, c2):
            ref[i, pl.ds(k * LANES, LANES)] = jnp.full((LANES,), val, jnp.float32)
            return c2
        return lax.fori_loop(0, ncols // LANES, col, carry)
    lax.fori_loop(0, nrows, row, 0)


@functools.partial(
    pl.kernel,
    out_type=jax.ShapeDtypeStruct((NP, D), jnp.float32),
    mesh=_MESH,
    scratch_types=(
        pltpu.VMEM((RW // KE, KE), jnp.int32),
        pltpu.VMEM((KE, D), jnp.float32),
        pltpu.SemaphoreType.DMA,
    ),
)
def _sc_embed(inf_hbm, emb_hbm, h0_hbm, ifbuf, erows, sem):
    c = lax.axis_index("c")
    s = lax.axis_index("s")
    w = c * NS + s
    pltpu.sync_copy(inf_hbm.at[w], ifbuf)

    def emb_step(j, carry):
        pltpu.async_copy(emb_hbm.at[ifbuf.at[j]], erows, sem).wait()
        pltpu.sync_copy(erows, h0_hbm.at[pl.ds(w * RW + j * KE, KE)])
        return carry
    lax.fori_loop(0, RW // KE, emb_step, 0)


@functools.partial(
    pl.kernel,
    out_type=(
        jax.ShapeDtypeStruct((NC, NP, D), jnp.float32),
        jax.ShapeDtypeStruct((NC, NP, D), jnp.float32),
    ),
    mesh=_MESH,
    scratch_types=(
        pltpu.VMEM((SBB, K), jnp.int32),
        pltpu.VMEM((SBB, K), jnp.int32),
        pltpu.VMEM((K, D), jnp.float32),
        pltpu.VMEM((K, D), jnp.float32),
        pltpu.VMEM((LANES,), jnp.int32),
        pltpu.VMEM_SHARED((NP, D), jnp.float32),
        pltpu.SemaphoreType.DMA,
        pltpu.SemaphoreType.DMA,
    ),
)
def _sc_segsum(tab_hbm, src_hbm, dst_hbm, flag_hbm, agg_hbm, deg_hbm,
               srcbuf, dstbuf, rows0, rows1, fvec, acc, sem0, sem1):
    c = lax.axis_index("c")
    s = lax.axis_index("s")
    w = c * NS + s

    def zero_acc():
        _fill_rows(rows0, K, D, 0.0)
        def z(i, carry):
            pltpu.sync_copy(rows0, acc.at[pl.ds(s * NT + i * K, K)])
            return carry
        lax.fori_loop(0, NT // K, z, 0)

    def read_acc(out_hbm):
        def o(i, carry):
            pltpu.sync_copy(acc.at[pl.ds(s * NT + i * K, K)], rows0)
            pltpu.sync_copy(rows0, out_hbm.at[c, pl.ds(s * NT + i * K, K)])
            return carry
        lax.fori_loop(0, NT // K, o, 0)

    zero_acc()
    pltpu.sync_copy(flag_hbm, fvec)
    plsc.subcore_barrier()

    def super_block(sb, carry):
        pltpu.sync_copy(src_hbm.at[w * NSB + sb], srcbuf)
        pltpu.sync_copy(dst_hbm.at[w * NSB + sb], dstbuf)
        pltpu.async_copy(tab_hbm.at[srcbuf.at[0]], rows0, sem0)
        def pipe(p, c2):
            j0 = 2 * p
            j1 = j0 + 1
            pltpu.make_async_copy(tab_hbm.at[srcbuf.at[j0]], rows0, sem0).wait()
            pltpu.async_copy(tab_hbm.at[srcbuf.at[j1]], rows1, sem1)
            pltpu.sync_copy(rows0, acc.at[dstbuf.at[j0]], add=True)
            pltpu.make_async_copy(tab_hbm.at[srcbuf.at[j1]], rows1, sem1).wait()
            @pl.when(j1 + 1 < SBB)
            def _():
                pltpu.async_copy(tab_hbm.at[srcbuf.at[j1 + 1]], rows0, sem0)
            pltpu.sync_copy(rows1, acc.at[dstbuf.at[j1]], add=True)
            return c2
        return lax.fori_loop(0, SBB // 2, pipe, carry)
    lax.fori_loop(0, NSB, super_block, 0)
    plsc.subcore_barrier()
    read_acc(agg_hbm)

    do_deg = fvec[...][0]
    @pl.when(do_deg > 0)
    def _():
        plsc.subcore_barrier()
        zero_acc()
        plsc.subcore_barrier()
        _fill_rows(rows0, K, D, 1.0)
        def deg_sb(sb, carry):
            pltpu.sync_copy(dst_hbm.at[w * NSB + sb], dstbuf)
            def deg_step(j, c2):
                pltpu.sync_copy(rows0, acc.at[dstbuf.at[j]], add=True)
                return c2
            return lax.fori_loop(0, SBB, deg_step, carry)
        lax.fori_loop(0, NSB, deg_sb, 0)
        plsc.subcore_barrier()
        read_acc(deg_hbm)


_BLK = 1024
_GRID = NP // _BLK


def _tc_layer_body(h_ref, agg_ref, deg_ref, ws_ref, wn_ref, b_ref, out_ref):
    deg = deg_ref[0, :, 0] + deg_ref[1, :, 0]
    inv = 1.0 / jnp.maximum(deg, 1.0)
    hn = (agg_ref[0] + agg_ref[1]) * inv[:, None]
    acc = jnp.dot(h_ref[...], ws_ref[...], preferred_element_type=jnp.float32)
    acc = acc + jnp.dot(hn, wn_ref[...], preferred_element_type=jnp.float32)
    out_ref[...] = jnp.maximum(acc + b_ref[...], 0.0)


def _tc_layer(h, agg, deg, w_self, w_neigh, b):
    return pl.pallas_call(
        _tc_layer_body,
        grid=(_GRID,),
        in_specs=[
            pl.BlockSpec((_BLK, D), lambda i: (i, 0)),
            pl.BlockSpec((NC, _BLK, D), lambda i: (0, i, 0)),
            pl.BlockSpec((NC, _BLK, D), lambda i: (0, i, 0)),
            pl.BlockSpec((D, D), lambda i: (0, 0)),
            pl.BlockSpec((D, D), lambda i: (0, 0)),
            pl.BlockSpec((1, D), lambda i: (0, 0)),
        ],
        out_specs=pl.BlockSpec((_BLK, D), lambda i: (i, 0)),
        out_shape=jax.ShapeDtypeStruct((NP, D), jnp.float32),
    )(h, agg, deg, w_self, w_neigh, b.reshape(1, D))


def kernel(in_feat, edge_index, emb, W_self1, W_neigh1, b1, W_self2, W_neigh2, b2):
    src = edge_index[0].astype(jnp.int32).reshape(NW * NSB, SBB, K)
    dst = edge_index[1].astype(jnp.int32).reshape(NW * NSB, SBB, K)
    inf = jnp.concatenate(
        [in_feat.astype(jnp.int32), jnp.zeros((NP - N,), jnp.int32)]
    ).reshape(NW, RW // KE, KE)
    flag_on = jnp.ones((LANES,), jnp.int32)
    flag_off = jnp.zeros((LANES,), jnp.int32)

    h0 = _sc_embed(inf, emb)
    agg1, deg = _sc_segsum(h0, src, dst, flag_on)
    h1 = _tc_layer(h0, agg1, deg, W_self1, W_neigh1, b1)
    agg2, _unused = _sc_segsum(h1, src, dst, flag_off)
    h2 = _tc_layer(h1, agg2, deg, W_self2, W_neigh2, b2)
    return h2[:N]


# speedup vs baseline: 1.1147x; 1.1147x over previous
import functools

import jax
import jax.numpy as jnp
from jax import lax
from jax.experimental import pallas as pl
from jax.experimental.pallas import tpu as pltpu
from jax.experimental.pallas import tpu_sc as plsc

N = 10000
E = 320000
D = 128
NP = 10240
NC = 2
NS = 16
NW = NC * NS
EW = E // NW
K = 50
NB = EW // K
SBB = 40
NSB = NB // SBB
KE = 80
NT = NP // NS
RW = NP // NW
LANES = 16

_MESH = plsc.VectorSubcoreMesh(core_axis_name="c", subcore_axis_name="s")


def _fill_rows(ref, nrows, ncols, val):
    def row(i, carry):
        def col(k, c2):
            ref[i, pl.ds(k * LANES, LANES)] = jnp.full((LANES,), val, jnp.float32)
            return c2
        return lax.fori_loop(0, ncols // LANES, col, carry)
    lax.fori_loop(0, nrows, row, 0)


@functools.partial(
    pl.kernel,
    out_type=jax.ShapeDtypeStruct((NP, D), jnp.float32),
    mesh=_MESH,
    scratch_types=(
        pltpu.VMEM((RW // KE, KE), jnp.int32),
        pltpu.VMEM((KE, D), jnp.float32),
        pltpu.SemaphoreType.DMA,
    ),
)
def _sc_embed(inf_hbm, emb_hbm, h0_hbm, ifbuf, erows, sem):
    c = lax.axis_index("c")
    s = lax.axis_index("s")
    w = c * NS + s
    pltpu.sync_copy(inf_hbm.at[w], ifbuf)

    def emb_step(j, carry):
        pltpu.async_copy(emb_hbm.at[ifbuf.at[j]], erows, sem).wait()
        pltpu.sync_copy(erows, h0_hbm.at[pl.ds(w * RW + j * KE, KE)])
        return carry
    lax.fori_loop(0, RW // KE, emb_step, 0)


@functools.partial(
    pl.kernel,
    out_type=(
        jax.ShapeDtypeStruct((NC, NP, D), jnp.float32),
        jax.ShapeDtypeStruct((NC, NP, D), jnp.float32),
    ),
    mesh=_MESH,
    scratch_types=(
        pltpu.VMEM((SBB, K), jnp.int32),
        pltpu.VMEM((SBB, K), jnp.int32),
        pltpu.VMEM((K, D), jnp.float32),
        pltpu.VMEM((K, D), jnp.float32),
        pltpu.VMEM((LANES,), jnp.int32),
        pltpu.VMEM_SHARED((NP, D), jnp.float32),
        pltpu.SemaphoreType.DMA,
        pltpu.SemaphoreType.DMA,
        pltpu.SemaphoreType.DMA,
        pltpu.SemaphoreType.DMA,
    ),
)
def _sc_segsum(tab_hbm, src_hbm, dst_hbm, flag_hbm, agg_hbm, deg_hbm,
               srcbuf, dstbuf, rows0, rows1, fvec, acc, semg0, semg1, sems0, sems1):
    c = lax.axis_index("c")
    s = lax.axis_index("s")
    w = c * NS + s

    RC = 40

    def zero_acc():
        _fill_rows(rows0, RC, D, 0.0)
        def z(i, carry):
            pltpu.sync_copy(rows0.at[pl.ds(0, RC)],
                            acc.at[pl.ds(s * NT + i * RC, RC)])
            return carry
        lax.fori_loop(0, NT // RC, z, 0)

    def read_acc(out_hbm):
        def o(i, carry):
            pltpu.sync_copy(acc.at[pl.ds(s * NT + i * RC, RC)],
                            rows0.at[pl.ds(0, RC)])
            pltpu.sync_copy(rows0.at[pl.ds(0, RC)],
                            out_hbm.at[c, pl.ds(s * NT + i * RC, RC)])
            return carry
        lax.fori_loop(0, NT // RC, o, 0)

    zero_acc()
    pltpu.sync_copy(flag_hbm, fvec)
    plsc.subcore_barrier()

    def super_block(sb, carry):
        pltpu.sync_copy(src_hbm.at[w * NSB + sb], srcbuf)
        pltpu.sync_copy(dst_hbm.at[w * NSB + sb], dstbuf)
        pltpu.async_copy(tab_hbm.at[srcbuf.at[0]], rows0, semg0)
        def pipe(p, c2):
            j0 = 2 * p
            j1 = j0 + 1
            pltpu.make_async_copy(tab_hbm.at[srcbuf.at[j0]], rows0, semg0).wait()
            pltpu.async_copy(tab_hbm.at[srcbuf.at[j1]], rows1, semg1)
            s0 = pltpu.async_copy(rows0, acc.at[dstbuf.at[j0]], sems0, add=True)
            s0.wait()
            pltpu.make_async_copy(tab_hbm.at[srcbuf.at[j1]], rows1, semg1).wait()
            @pl.when(j1 + 1 < SBB)
            def _():
                pltpu.async_copy(tab_hbm.at[srcbuf.at[j1 + 1]], rows0, semg0)
            s1 = pltpu.async_copy(rows1, acc.at[dstbuf.at[j1]], sems1, add=True)
            s1.wait()
            return c2
        return lax.fori_loop(0, SBB // 2, pipe, carry)
    lax.fori_loop(0, NSB, super_block, 0)
    plsc.subcore_barrier()
    read_acc(agg_hbm)

    do_deg = fvec[...][0]
    @pl.when(do_deg > 0)
    def _():
        plsc.subcore_barrier()
        zero_acc()
        plsc.subcore_barrier()
        _fill_rows(rows0, K, D, 1.0)
        def deg_sb(sb, carry):
            pltpu.sync_copy(dst_hbm.at[w * NSB + sb], dstbuf)
            def deg_step(j, c2):
                pltpu.sync_copy(rows0, acc.at[dstbuf.at[j]], add=True)
                return c2
            return lax.fori_loop(0, SBB, deg_step, carry)
        lax.fori_loop(0, NSB, deg_sb, 0)
        plsc.subcore_barrier()
        read_acc(deg_hbm)


_BLK = 1024
_GRID = NP // _BLK


def _tc_layer_body(h_ref, agg_ref, deg_ref, ws_ref, wn_ref, b_ref, out_ref):
    deg = deg_ref[0, :, 0] + deg_ref[1, :, 0]
    inv = 1.0 / jnp.maximum(deg, 1.0)
    hn = (agg_ref[0] + agg_ref[1]) * inv[:, None]
    acc = jnp.dot(h_ref[...], ws_ref[...], preferred_element_type=jnp.float32)
    acc = acc + jnp.dot(hn, wn_ref[...], preferred_element_type=jnp.float32)
    out_ref[...] = jnp.maximum(acc + b_ref[...], 0.0)


def _tc_layer(h, agg, deg, w_self, w_neigh, b):
    return pl.pallas_call(
        _tc_layer_body,
        grid=(_GRID,),
        in_specs=[
            pl.BlockSpec((_BLK, D), lambda i: (i, 0)),
            pl.BlockSpec((NC, _BLK, D), lambda i: (0, i, 0)),
            pl.BlockSpec((NC, _BLK, D), lambda i: (0, i, 0)),
            pl.BlockSpec((D, D), lambda i: (0, 0)),
            pl.BlockSpec((D, D), lambda i: (0, 0)),
            pl.BlockSpec((1, D), lambda i: (0, 0)),
        ],
        out_specs=pl.BlockSpec((_BLK, D), lambda i: (i, 0)),
        out_shape=jax.ShapeDtypeStruct((NP, D), jnp.float32),
    )(h, agg, deg, w_self, w_neigh, b.reshape(1, D))


def kernel(in_feat, edge_index, emb, W_self1, W_neigh1, b1, W_self2, W_neigh2, b2):
    src = edge_index[0].astype(jnp.int32).reshape(NW * NSB, SBB, K)
    dst = edge_index[1].astype(jnp.int32).reshape(NW * NSB, SBB, K)
    inf = jnp.concatenate(
        [in_feat.astype(jnp.int32), jnp.zeros((NP - N,), jnp.int32)]
    ).reshape(NW, RW // KE, KE)
    flag_on = jnp.ones((LANES,), jnp.int32)
    flag_off = jnp.zeros((LANES,), jnp.int32)

    h0 = _sc_embed(inf, emb)
    agg1, deg = _sc_segsum(h0, src, dst, flag_on)
    h1 = _tc_layer(h0, agg1, deg, W_self1, W_neigh1, b1)
    agg2, _unused = _sc_segsum(h1, src, dst, flag_off)
    h2 = _tc_layer(h1, agg2, deg, W_self2, W_neigh2, b2)
    return h2[:N]


# speedup vs baseline: 1.4226x; 1.2762x over previous
import functools

import jax
import jax.numpy as jnp
from jax import lax
from jax.experimental import pallas as pl
from jax.experimental.pallas import tpu as pltpu
from jax.experimental.pallas import tpu_sc as plsc

N = 10000
E = 320000
D = 128
NP = 10240
NC = 2
NS = 16
NW = NC * NS
EW = E // NW
K = 100
NB = EW // K
SBB = 20
NSB = NB // SBB
KE = 80
NT = NP // NS
RW = NP // NW
LANES = 16

_MESH = plsc.VectorSubcoreMesh(core_axis_name="c", subcore_axis_name="s")


def _fill_rows(ref, nrows, ncols, val):
    def row(i, carry):
        def col(k, c2):
            ref[i, pl.ds(k * LANES, LANES)] = jnp.full((LANES,), val, jnp.float32)
            return c2
        return lax.fori_loop(0, ncols // LANES, col, carry)
    lax.fori_loop(0, nrows, row, 0)


@functools.partial(
    pl.kernel,
    out_type=jax.ShapeDtypeStruct((NP, D), jnp.float32),
    mesh=_MESH,
    scratch_types=(
        pltpu.VMEM((RW // KE, KE), jnp.int32),
        pltpu.VMEM((KE, D), jnp.float32),
        pltpu.SemaphoreType.DMA,
    ),
)
def _sc_embed(inf_hbm, emb_hbm, h0_hbm, ifbuf, erows, sem):
    c = lax.axis_index("c")
    s = lax.axis_index("s")
    w = c * NS + s
    pltpu.sync_copy(inf_hbm.at[w], ifbuf)

    def emb_step(j, carry):
        pltpu.async_copy(emb_hbm.at[ifbuf.at[j]], erows, sem).wait()
        pltpu.sync_copy(erows, h0_hbm.at[pl.ds(w * RW + j * KE, KE)])
        return carry
    lax.fori_loop(0, RW // KE, emb_step, 0)


@functools.partial(
    pl.kernel,
    out_type=(
        jax.ShapeDtypeStruct((NC, NP, D), jnp.float32),
        jax.ShapeDtypeStruct((NC, NP, D), jnp.float32),
    ),
    mesh=_MESH,
    scratch_types=(
        pltpu.VMEM((SBB, K), jnp.int32),
        pltpu.VMEM((SBB, K), jnp.int32),
        pltpu.VMEM((K, D), jnp.float32),
        pltpu.VMEM((K, D), jnp.float32),
        pltpu.VMEM((LANES,), jnp.int32),
        pltpu.VMEM_SHARED((NP, D), jnp.float32),
        pltpu.SemaphoreType.DMA,
        pltpu.SemaphoreType.DMA,
        pltpu.SemaphoreType.DMA,
        pltpu.SemaphoreType.DMA,
    ),
)
def _sc_segsum(tab_hbm, src_hbm, dst_hbm, flag_hbm, agg_hbm, deg_hbm,
               srcbuf, dstbuf, rows0, rows1, fvec, acc, semg0, semg1, sems0, sems1):
    c = lax.axis_index("c")
    s = lax.axis_index("s")
    w = c * NS + s

    RC = 40

    def zero_acc():
        _fill_rows(rows0, RC, D, 0.0)
        def z(i, carry):
            pltpu.sync_copy(rows0.at[pl.ds(0, RC)],
                            acc.at[pl.ds(s * NT + i * RC, RC)])
            return carry
        lax.fori_loop(0, NT // RC, z, 0)

    def read_acc(out_hbm):
        def o(i, carry):
            pltpu.sync_copy(acc.at[pl.ds(s * NT + i * RC, RC)],
                            rows0.at[pl.ds(0, RC)])
            pltpu.sync_copy(rows0.at[pl.ds(0, RC)],
                            out_hbm.at[c, pl.ds(s * NT + i * RC, RC)])
            return carry
        lax.fori_loop(0, NT // RC, o, 0)

    zero_acc()
    pltpu.sync_copy(flag_hbm, fvec)
    plsc.subcore_barrier()

    def super_block(sb, carry):
        pltpu.sync_copy(src_hbm.at[w * NSB + sb], srcbuf)
        pltpu.sync_copy(dst_hbm.at[w * NSB + sb], dstbuf)
        pltpu.async_copy(tab_hbm.at[srcbuf.at[0]], rows0, semg0)
        def pipe(p, c2):
            j0 = 2 * p
            j1 = j0 + 1
            pltpu.make_async_copy(tab_hbm.at[srcbuf.at[j0]], rows0, semg0).wait()
            pltpu.async_copy(tab_hbm.at[srcbuf.at[j1]], rows1, semg1)
            s0 = pltpu.async_copy(rows0, acc.at[dstbuf.at[j0]], sems0, add=True)
            s0.wait()
            pltpu.make_async_copy(tab_hbm.at[srcbuf.at[j1]], rows1, semg1).wait()
            @pl.when(j1 + 1 < SBB)
            def _():
                pltpu.async_copy(tab_hbm.at[srcbuf.at[j1 + 1]], rows0, semg0)
            s1 = pltpu.async_copy(rows1, acc.at[dstbuf.at[j1]], sems1, add=True)
            s1.wait()
            return c2
        return lax.fori_loop(0, SBB // 2, pipe, carry)
    lax.fori_loop(0, NSB, super_block, 0)
    plsc.subcore_barrier()
    read_acc(agg_hbm)

    do_deg = fvec[...][0]
    @pl.when(do_deg > 0)
    def _():
        plsc.subcore_barrier()
        zero_acc()
        plsc.subcore_barrier()
        _fill_rows(rows0, K, D, 1.0)
        def deg_sb(sb, carry):
            pltpu.sync_copy(dst_hbm.at[w * NSB + sb], dstbuf)
            def deg_step(j, c2):
                pltpu.sync_copy(rows0, acc.at[dstbuf.at[j]], add=True)
                return c2
            return lax.fori_loop(0, SBB, deg_step, carry)
        lax.fori_loop(0, NSB, deg_sb, 0)
        plsc.subcore_barrier()
        read_acc(deg_hbm)


_BLK = 1024
_GRID = NP // _BLK


def _tc_layer_body(h_ref, agg_ref, deg_ref, ws_ref, wn_ref, b_ref, out_ref):
    deg = deg_ref[0, :, 0] + deg_ref[1, :, 0]
    inv = 1.0 / jnp.maximum(deg, 1.0)
    hn = (agg_ref[0] + agg_ref[1]) * inv[:, None]
    acc = jnp.dot(h_ref[...], ws_ref[...], preferred_element_type=jnp.float32)
    acc = acc + jnp.dot(hn, wn_ref[...], preferred_element_type=jnp.float32)
    out_ref[...] = jnp.maximum(acc + b_ref[...], 0.0)


def _tc_layer(h, agg, deg, w_self, w_neigh, b):
    return pl.pallas_call(
        _tc_layer_body,
        grid=(_GRID,),
        in_specs=[
            pl.BlockSpec((_BLK, D), lambda i: (i, 0)),
            pl.BlockSpec((NC, _BLK, D), lambda i: (0, i, 0)),
            pl.BlockSpec((NC, _BLK, D), lambda i: (0, i, 0)),
            pl.BlockSpec((D, D), lambda i: (0, 0)),
            pl.BlockSpec((D, D), lambda i: (0, 0)),
            pl.BlockSpec((1, D), lambda i: (0, 0)),
        ],
        out_specs=pl.BlockSpec((_BLK, D), lambda i: (i, 0)),
        out_shape=jax.ShapeDtypeStruct((NP, D), jnp.float32),
    )(h, agg, deg, w_self, w_neigh, b.reshape(1, D))


def kernel(in_feat, edge_index, emb, W_self1, W_neigh1, b1, W_self2, W_neigh2, b2):
    src = edge_index[0].astype(jnp.int32).reshape(NW * NSB, SBB, K)
    dst = edge_index[1].astype(jnp.int32).reshape(NW * NSB, SBB, K)
    inf = jnp.concatenate(
        [in_feat.astype(jnp.int32), jnp.zeros((NP - N,), jnp.int32)]
    ).reshape(NW, RW // KE, KE)
    flag_on = jnp.ones((LANES,), jnp.int32)
    flag_off = jnp.zeros((LANES,), jnp.int32)

    h0 = _sc_embed(inf, emb)
    agg1, deg = _sc_segsum(h0, src, dst, flag_on)
    h1 = _tc_layer(h0, agg1, deg, W_self1, W_neigh1, b1)
    agg2, _unused = _sc_segsum(h1, src, dst, flag_off)
    h2 = _tc_layer(h1, agg2, deg, W_self2, W_neigh2, b2)
    return h2[:N]


# speedup vs baseline: 1.4338x; 1.0079x over previous
import functools

import jax
import jax.numpy as jnp
from jax import lax
from jax.experimental import pallas as pl
from jax.experimental.pallas import tpu as pltpu
from jax.experimental.pallas import tpu_sc as plsc

N = 10000
E = 320000
D = 128
NP = 10240
NC = 2
NS = 16
NW = NC * NS
EW = E // NW
K = 100
NB = EW // K
SBB = 20
NSB = NB // SBB
KE = 80
NT = NP // NS
RW = NP // NW
LANES = 16

_MESH = plsc.VectorSubcoreMesh(core_axis_name="c", subcore_axis_name="s")


def _fill_rows(ref, nrows, ncols, val):
    def row(i, carry):
        def col(k, c2):
            ref[i, pl.ds(k * LANES, LANES)] = jnp.full((LANES,), val, jnp.float32)
            return c2
        return lax.fori_loop(0, ncols // LANES, col, carry)
    lax.fori_loop(0, nrows, row, 0)


@functools.partial(
    pl.kernel,
    out_type=jax.ShapeDtypeStruct((NP, D), jnp.float32),
    mesh=_MESH,
    scratch_types=(
        pltpu.VMEM((RW // KE, KE), jnp.int32),
        pltpu.VMEM((KE, D), jnp.float32),
        pltpu.SemaphoreType.DMA,
    ),
)
def _sc_embed(inf_hbm, emb_hbm, h0_hbm, ifbuf, erows, sem):
    c = lax.axis_index("c")
    s = lax.axis_index("s")
    w = c * NS + s
    pltpu.sync_copy(inf_hbm.at[w], ifbuf)

    def emb_step(j, carry):
        pltpu.async_copy(emb_hbm.at[ifbuf.at[j]], erows, sem).wait()
        pltpu.sync_copy(erows, h0_hbm.at[pl.ds(w * RW + j * KE, KE)])
        return carry
    lax.fori_loop(0, RW // KE, emb_step, 0)


@functools.partial(
    pl.kernel,
    out_type=(
        jax.ShapeDtypeStruct((NC, NP, D), jnp.float32),
        jax.ShapeDtypeStruct((NC, NP, D), jnp.float32),
    ),
    mesh=_MESH,
    scratch_types=(
        pltpu.VMEM((SBB, K), jnp.int32),
        pltpu.VMEM((SBB, K), jnp.int32),
        pltpu.VMEM((K, D), jnp.float32),
        pltpu.VMEM((K, D), jnp.float32),
        pltpu.VMEM((LANES,), jnp.int32),
        pltpu.VMEM_SHARED((NP, D), jnp.float32),
        pltpu.SemaphoreType.DMA,
        pltpu.SemaphoreType.DMA,
        pltpu.SemaphoreType.DMA,
        pltpu.SemaphoreType.DMA,
    ),
)
def _sc_segsum(tab_hbm, src_hbm, dst_hbm, flag_hbm, agg_hbm, deg_hbm,
               srcbuf, dstbuf, rows0, rows1, fvec, acc, semg0, semg1, sems0, sems1):
    c = lax.axis_index("c")
    s = lax.axis_index("s")
    w = c * NS + s

    RC = 80

    def zero_acc():
        _fill_rows(rows0, RC, D, 0.0)
        def z(i, carry):
            pltpu.sync_copy(rows0.at[pl.ds(0, RC)],
                            acc.at[pl.ds(s * NT + i * RC, RC)])
            return carry
        lax.fori_loop(0, NT // RC, z, 0)

    def read_acc(out_hbm):
        def o(i, carry):
            pltpu.sync_copy(acc.at[pl.ds(s * NT + i * RC, RC)],
                            rows0.at[pl.ds(0, RC)])
            pltpu.sync_copy(rows0.at[pl.ds(0, RC)],
                            out_hbm.at[c, pl.ds(s * NT + i * RC, RC)])
            return carry
        lax.fori_loop(0, NT // RC, o, 0)

    zero_acc()
    pltpu.sync_copy(flag_hbm, fvec)
    plsc.subcore_barrier()

    def super_block(sb, carry):
        pltpu.sync_copy(src_hbm.at[w * NSB + sb], srcbuf)
        pltpu.sync_copy(dst_hbm.at[w * NSB + sb], dstbuf)
        pltpu.async_copy(tab_hbm.at[srcbuf.at[0]], rows0, semg0)
        def pipe(p, c2):
            j0 = 2 * p
            j1 = j0 + 1
            pltpu.make_async_copy(tab_hbm.at[srcbuf.at[j0]], rows0, semg0).wait()
            pltpu.async_copy(tab_hbm.at[srcbuf.at[j1]], rows1, semg1)
            s0 = pltpu.async_copy(rows0, acc.at[dstbuf.at[j0]], sems0, add=True)
            s0.wait()
            pltpu.make_async_copy(tab_hbm.at[srcbuf.at[j1]], rows1, semg1).wait()
            @pl.when(j1 + 1 < SBB)
            def _():
                pltpu.async_copy(tab_hbm.at[srcbuf.at[j1 + 1]], rows0, semg0)
            s1 = pltpu.async_copy(rows1, acc.at[dstbuf.at[j1]], sems1, add=True)
            s1.wait()
            return c2
        return lax.fori_loop(0, SBB // 2, pipe, carry)
    lax.fori_loop(0, NSB, super_block, 0)
    plsc.subcore_barrier()
    read_acc(agg_hbm)

    do_deg = fvec[...][0]
    @pl.when(do_deg > 0)
    def _():
        plsc.subcore_barrier()
        zero_acc()
        plsc.subcore_barrier()
        _fill_rows(rows0, K, D, 1.0)
        def deg_sb(sb, carry):
            pltpu.sync_copy(dst_hbm.at[w * NSB + sb], dstbuf)
            def deg_step(j, c2):
                pltpu.sync_copy(rows0, acc.at[dstbuf.at[j]], add=True)
                return c2
            return lax.fori_loop(0, SBB, deg_step, carry)
        lax.fori_loop(0, NSB, deg_sb, 0)
        plsc.subcore_barrier()
        read_acc(deg_hbm)


_BLK = 1024
_GRID = NP // _BLK


def _tc_layer_body(h_ref, agg_ref, deg_ref, ws_ref, wn_ref, b_ref, out_ref):
    deg = deg_ref[0, :, 0] + deg_ref[1, :, 0]
    inv = 1.0 / jnp.maximum(deg, 1.0)
    hn = (agg_ref[0] + agg_ref[1]) * inv[:, None]
    acc = jnp.dot(h_ref[...], ws_ref[...], preferred_element_type=jnp.float32)
    acc = acc + jnp.dot(hn, wn_ref[...], preferred_element_type=jnp.float32)
    out_ref[...] = jnp.maximum(acc + b_ref[...], 0.0)


def _tc_layer(h, agg, deg, w_self, w_neigh, b):
    return pl.pallas_call(
        _tc_layer_body,
        grid=(_GRID,),
        in_specs=[
            pl.BlockSpec((_BLK, D), lambda i: (i, 0)),
            pl.BlockSpec((NC, _BLK, D), lambda i: (0, i, 0)),
            pl.BlockSpec((NC, _BLK, D), lambda i: (0, i, 0)),
            pl.BlockSpec((D, D), lambda i: (0, 0)),
            pl.BlockSpec((D, D), lambda i: (0, 0)),
            pl.BlockSpec((1, D), lambda i: (0, 0)),
        ],
        out_specs=pl.BlockSpec((_BLK, D), lambda i: (i, 0)),
        out_shape=jax.ShapeDtypeStruct((NP, D), jnp.float32),
    )(h, agg, deg, w_self, w_neigh, b.reshape(1, D))


def kernel(in_feat, edge_index, emb, W_self1, W_neigh1, b1, W_self2, W_neigh2, b2):
    src = edge_index[0].astype(jnp.int32).reshape(NW * NSB, SBB, K)
    dst = edge_index[1].astype(jnp.int32).reshape(NW * NSB, SBB, K)
    inf = jnp.concatenate(
        [in_feat.astype(jnp.int32), jnp.zeros((NP - N,), jnp.int32)]
    ).reshape(NW, RW // KE, KE)
    flag_on = jnp.ones((LANES,), jnp.int32)
    flag_off = jnp.zeros((LANES,), jnp.int32)

    h0 = _sc_embed(inf, emb)
    agg1, deg = _sc_segsum(h0, src, dst, flag_on)
    h1 = _tc_layer(h0, agg1, deg, W_self1, W_neigh1, b1)
    agg2, _unused = _sc_segsum(h1, src, dst, flag_off)
    h2 = _tc_layer(h1, agg2, deg, W_self2, W_neigh2, b2)
    return h2[:N]
